# baseline (speedup 1.0000x reference)
import functools

import jax
import jax.numpy as jnp
from jax.experimental import pallas as pl
from jax.experimental.pallas import tpu as pltpu


def _ffn_body(x_ref, w1_ref, w2_ref, sgw_ref, out_ref, sg_ref, *, bi):
    j = pl.program_id(1)

    x = x_ref[...]

    @pl.when(j == 0)
    def _():
        prod = x.astype(jnp.float32) * sgw_ref[...].astype(jnp.float32)
        sg_ref[...] = jax.nn.sigmoid(jnp.sum(prod, axis=1, keepdims=True))

    dims = (((1,), (1,)), ((), ()))
    y = jax.lax.dot_general(x, w1_ref[...], dims,
                            preferred_element_type=jnp.float32)
    g = y[:, :bi]
    u = y[:, bi:2 * bi]
    s = y[:, 2 * bi:]

    a1 = g * jax.nn.sigmoid(g) * u
    a2 = s * jax.nn.sigmoid(s) * sg_ref[...]
    ab = jnp.concatenate([a1, a2], axis=1).astype(jnp.bfloat16)

    contrib = jax.lax.dot_general(ab, w2_ref[...], dims,
                                  preferred_element_type=jnp.float32)

    @pl.when(j == 0)
    def _():
        out_ref[...] = contrib

    @pl.when(j > 0)
    def _():
        out_ref[...] += contrib


def kernel(hidden_states, w13, w2, gate, shared_w1, shared_w2, shared_gate_w):
    del gate
    bsz, seq_len, hidden = hidden_states.shape
    n_tokens = bsz * seq_len
    inter = shared_w1.shape[0]

    bm = 1024 if n_tokens % 1024 == 0 else n_tokens
    bi = 512 if inter % 512 == 0 else inter
    num_i = n_tokens // bm
    num_j = inter // bi

    x = hidden_states.reshape(n_tokens, hidden).astype(jnp.bfloat16)

    w1cat = jnp.concatenate(
        [w13.astype(jnp.bfloat16), shared_w1.astype(jnp.bfloat16)], axis=0)
    w1cat = (w1cat.reshape(3, num_j, bi, hidden)
             .transpose(1, 0, 2, 3)
             .reshape(3 * inter, hidden))

    w2cat = jnp.stack(
        [w2.astype(jnp.bfloat16).reshape(hidden, num_j, bi),
         shared_w2.astype(jnp.bfloat16).reshape(hidden, num_j, bi)], axis=2)
    w2cat = w2cat.reshape(hidden, 2 * inter)

    sgw_b = shared_gate_w.astype(jnp.bfloat16)

    out = pl.pallas_call(
        functools.partial(_ffn_body, bi=bi),
        grid=(num_i, num_j),
        in_specs=[
            pl.BlockSpec((bm, hidden), lambda i, j: (i, 0)),
            pl.BlockSpec((3 * bi, hidden), lambda i, j: (j, 0)),
            pl.BlockSpec((hidden, 2 * bi), lambda i, j: (0, j)),
            pl.BlockSpec((1, hidden), lambda i, j: (0, 0)),
        ],
        out_specs=pl.BlockSpec((bm, hidden), lambda i, j: (i, 0)),
        out_shape=jax.ShapeDtypeStruct((n_tokens, hidden), jnp.float32),
        scratch_shapes=[pltpu.VMEM((bm, 1), jnp.float32)],
        compiler_params=pltpu.CompilerParams(
            dimension_semantics=("parallel", "arbitrary")),
    )(x, w1cat, w2cat, sgw_b)

    return out.reshape(bsz, seq_len, hidden)


# speedup vs baseline: 1.6748x; 1.6748x over previous
import functools

import jax
import jax.numpy as jnp
from jax.experimental import pallas as pl
from jax.experimental.pallas import tpu as pltpu


def _ffn_body(x_ref, w1g_ref, w1u_ref, w1s_ref, w2_ref, sw2_ref, sgw_ref,
              out_ref, sg_ref):
    j = pl.program_id(1)

    x = x_ref[...]

    @pl.when(j == 0)
    def _():
        prod = x.astype(jnp.float32) * sgw_ref[...].astype(jnp.float32)
        sg_ref[...] = jax.nn.sigmoid(jnp.sum(prod, axis=1, keepdims=True))

    dims = (((1,), (1,)), ((), ()))
    g = jax.lax.dot_general(x, w1g_ref[...], dims,
                            preferred_element_type=jnp.float32)
    u = jax.lax.dot_general(x, w1u_ref[...], dims,
                            preferred_element_type=jnp.float32)
    s = jax.lax.dot_general(x, w1s_ref[...], dims,
                            preferred_element_type=jnp.float32)

    a1 = (g * jax.nn.sigmoid(g) * u).astype(jnp.bfloat16)
    a2 = (s * jax.nn.sigmoid(s) * sg_ref[...]).astype(jnp.bfloat16)

    contrib = jax.lax.dot_general(a1, w2_ref[...].astype(jnp.bfloat16), dims,
                                  preferred_element_type=jnp.float32)
    contrib += jax.lax.dot_general(a2, sw2_ref[...].astype(jnp.bfloat16), dims,
                                   preferred_element_type=jnp.float32)

    @pl.when(j == 0)
    def _():
        out_ref[...] = contrib

    @pl.when(j > 0)
    def _():
        out_ref[...] += contrib


def kernel(hidden_states, w13, w2, gate, shared_w1, shared_w2, shared_gate_w):
    del gate
    bsz, seq_len, hidden = hidden_states.shape
    n_tokens = bsz * seq_len
    inter = shared_w1.shape[0]

    x = hidden_states.reshape(n_tokens, hidden).astype(jnp.bfloat16)
    w13_b = w13.astype(jnp.bfloat16)
    sw1_b = shared_w1.astype(jnp.bfloat16)
    sgw_b = shared_gate_w.astype(jnp.bfloat16)

    bm = 1024 if n_tokens % 1024 == 0 else n_tokens
    bi = 512 if inter % 512 == 0 else inter
    num_i = n_tokens // bm
    num_j = inter // bi

    out = pl.pallas_call(
        functools.partial(_ffn_body),
        grid=(num_i, num_j),
        in_specs=[
            pl.BlockSpec((bm, hidden), lambda i, j: (i, 0)),
            pl.BlockSpec((bi, hidden), lambda i, j: (j, 0)),
            pl.BlockSpec((bi, hidden),
                         lambda i, j, nj=num_j: (j + nj, 0)),
            pl.BlockSpec((bi, hidden), lambda i, j: (j, 0)),
            pl.BlockSpec((hidden, bi), lambda i, j: (0, j)),
            pl.BlockSpec((hidden, bi), lambda i, j: (0, j)),
            pl.BlockSpec((1, hidden), lambda i, j: (0, 0)),
        ],
        out_specs=pl.BlockSpec((bm, hidden), lambda i, j: (i, 0)),
        out_shape=jax.ShapeDtypeStruct((n_tokens, hidden), jnp.float32),
        scratch_shapes=[pltpu.VMEM((bm, 1), jnp.float32)],
        compiler_params=pltpu.CompilerParams(
            dimension_semantics=("parallel", "arbitrary"),
            vmem_limit_bytes=64 * 1024 * 1024),
    )(x, w13_b, w13_b, sw1_b, w2, shared_w2, sgw_b)

    return out.reshape(bsz, seq_len, hidden)


# speedup vs baseline: 1.7394x; 1.0386x over previous
import functools

import jax
import jax.numpy as jnp
from jax.experimental import pallas as pl
from jax.experimental.pallas import tpu as pltpu


def _ffn_body(x_ref, w1g_ref, w1u_ref, w1s_ref, w2_ref, sw2_ref, sg_ref,
              out_ref):
    j = pl.program_id(1)

    x = x_ref[...]

    dims = (((1,), (1,)), ((), ()))
    g = jax.lax.dot_general(x, w1g_ref[...], dims,
                            preferred_element_type=jnp.float32)
    u = jax.lax.dot_general(x, w1u_ref[...], dims,
                            preferred_element_type=jnp.float32)
    s = jax.lax.dot_general(x, w1s_ref[...], dims,
                            preferred_element_type=jnp.float32)

    a1 = (g * jax.nn.sigmoid(g) * u).astype(jnp.bfloat16)
    a2 = (s * jax.nn.sigmoid(s) * sg_ref[...]).astype(jnp.bfloat16)

    contrib = jax.lax.dot_general(a1, w2_ref[...].astype(jnp.bfloat16), dims,
                                  preferred_element_type=jnp.float32)
    contrib += jax.lax.dot_general(a2, sw2_ref[...].astype(jnp.bfloat16), dims,
                                   preferred_element_type=jnp.float32)

    @pl.when(j == 0)
    def _():
        out_ref[...] = contrib

    @pl.when(j > 0)
    def _():
        out_ref[...] += contrib


def kernel(hidden_states, w13, w2, gate, shared_w1, shared_w2, shared_gate_w):
    del gate
    bsz, seq_len, hidden = hidden_states.shape
    n_tokens = bsz * seq_len
    inter = shared_w1.shape[0]

    xf = hidden_states.reshape(n_tokens, hidden)
    x = xf.astype(jnp.bfloat16)
    w13_b = w13.astype(jnp.bfloat16)
    sw1_b = shared_w1.astype(jnp.bfloat16)
    sg = jax.nn.sigmoid(xf @ shared_gate_w.T)

    bm = 1024 if n_tokens % 1024 == 0 else n_tokens
    bi = 512 if inter % 512 == 0 else inter
    num_i = n_tokens // bm
    num_j = inter // bi

    out = pl.pallas_call(
        functools.partial(_ffn_body),
        grid=(num_i, num_j),
        in_specs=[
            pl.BlockSpec((bm, hidden), lambda i, j: (i, 0)),
            pl.BlockSpec((bi, hidden), lambda i, j: (j, 0)),
            pl.BlockSpec((bi, hidden),
                         lambda i, j, nj=num_j: (j + nj, 0)),
            pl.BlockSpec((bi, hidden), lambda i, j: (j, 0)),
            pl.BlockSpec((hidden, bi), lambda i, j: (0, j)),
            pl.BlockSpec((hidden, bi), lambda i, j: (0, j)),
            pl.BlockSpec((bm, 1), lambda i, j: (i, 0)),
        ],
        out_specs=pl.BlockSpec((bm, hidden), lambda i, j: (i, 0)),
        out_shape=jax.ShapeDtypeStruct((n_tokens, hidden), jnp.float32),
        compiler_params=pltpu.CompilerParams(
            dimension_semantics=("parallel", "arbitrary"),
            vmem_limit_bytes=64 * 1024 * 1024),
    )(x, w13_b, w13_b, sw1_b, w2, shared_w2, sg)

    return out.reshape(bsz, seq_len, hidden)
